# baseline (speedup 1.0000x reference)
import functools

import jax
import jax.numpy as jnp
from jax import lax
from jax.experimental import pallas as pl
from jax.experimental.pallas import tpu as pltpu
from jax.experimental.pallas import tpu_sc as plsc

_K = 1024
_D = 64
_N = 32768
_RBLK = 128

_NC = 2
_NS = 16
_NW = _NC * _NS
_RPW = _N // _NW
_CH = 128
_L = 16


def _bins_body(a_ref, c_ref, cn_ref, bins_ref):
    a = a_ref[...]
    c = c_ref[...]
    prod = lax.dot_general(
        a, c,
        dimension_numbers=(((1,), (1,)), ((), ())),
        preferred_element_type=jnp.float32,
    )
    sq = jnp.sum(a * a, axis=1, keepdims=True) - 2.0 * prod + cn_ref[...]
    dists = jnp.sqrt(jnp.maximum(sq, 0.0))
    m = jnp.min(dists, axis=1, keepdims=True)
    idx = lax.broadcasted_iota(jnp.int32, dists.shape, 1).astype(jnp.float32)
    bins_ref[...] = jnp.min(jnp.where(dists <= m, idx, float(_K)), axis=1).astype(jnp.int32)


def _compute_bins(a, centroids):
    grid = _N // _RBLK
    cnorm = jnp.sum(centroids * centroids, axis=1)[None, :]
    return pl.pallas_call(
        _bins_body,
        grid=(grid,),
        in_specs=[
            pl.BlockSpec((_RBLK, _D), lambda i: (i, 0)),
            pl.BlockSpec((_K, _D), lambda i: (0, 0)),
            pl.BlockSpec((1, _K), lambda i: (0, 0)),
        ],
        out_specs=pl.BlockSpec((_RBLK,), lambda i: (i,)),
        out_shape=jax.ShapeDtypeStruct((_N,), jnp.int32),
    )(a, centroids, cnorm)


def _sc_residual_body(bins_hbm, act_hbm, ctr_hbm, out_hbm, idx_v, act_v, ctr_v, sem):
    wid = lax.axis_index("s") * _NC + lax.axis_index("c")
    base = wid * _RPW
    pltpu.sync_copy(bins_hbm.at[pl.ds(base, _RPW)], idx_v)

    def chunk(t, carry):
        rb = base + t * _CH
        pltpu.sync_copy(act_hbm.at[pl.ds(rb, _CH)], act_v)
        pltpu.async_copy(ctr_hbm.at[idx_v.at[pl.ds(t * _CH, _CH)]], ctr_v, sem).wait()

        def row(r, c2):
            for j in range(_D // _L):
                sl = pl.ds(j * _L, _L)
                act_v[r, sl] = act_v[r, sl] - ctr_v[r, sl]
            return c2

        lax.fori_loop(0, _CH, row, 0)
        pltpu.sync_copy(act_v, out_hbm.at[pl.ds(rb, _CH)])
        return carry

    lax.fori_loop(0, _RPW // _CH, chunk, 0)


@functools.lru_cache(maxsize=1)
def _sc_residual():
    return pl.kernel(
        _sc_residual_body,
        mesh=plsc.VectorSubcoreMesh(core_axis_name="c", subcore_axis_name="s"),
        out_type=jax.ShapeDtypeStruct((_N, _D), jnp.float32),
        scratch_types=[
            pltpu.VMEM((_RPW,), jnp.int32),
            pltpu.VMEM((_CH, _D), jnp.float32),
            pltpu.VMEM((_CH, 128), jnp.float32),
            pltpu.SemaphoreType.DMA,
        ],
    )


def kernel(action, centroids):
    B, T, D = action.shape
    a = action.reshape(B * T, D)
    bins = _compute_bins(a, centroids)
    cpad = jnp.pad(centroids, ((0, 0), (0, 128 - _D)))
    resid = _sc_residual()(bins, a, cpad)
    return bins.reshape(B, T, 1).astype(jnp.int64), resid.reshape(B, T, D)


# speedup vs baseline: 1.4764x; 1.4764x over previous
import functools

import jax
import jax.numpy as jnp
from jax import lax
from jax.experimental import pallas as pl
from jax.experimental.pallas import tpu as pltpu
from jax.experimental.pallas import tpu_sc as plsc

_K = 1024
_D = 64
_N = 32768
_RBLK = 1024

_NC = 2
_NS = 16
_NW = _NC * _NS
_RPW = _N // _NW
_CH = 128
_L = 16


def _bins_body(a_ref, c_ref, cn_ref, bins_ref):
    a = a_ref[...]
    c = c_ref[...]
    prod = lax.dot_general(
        a, c,
        dimension_numbers=(((1,), (1,)), ((), ())),
        preferred_element_type=jnp.float32,
    )
    sq = jnp.sum(a * a, axis=1, keepdims=True) - 2.0 * prod + cn_ref[...]
    dists = jnp.sqrt(jnp.maximum(sq, 0.0))
    m = jnp.min(dists, axis=1, keepdims=True)
    idx = lax.broadcasted_iota(jnp.int32, dists.shape, 1).astype(jnp.float32)
    bins_ref[...] = jnp.min(jnp.where(dists <= m, idx, float(_K)), axis=1).astype(jnp.int32)


def _compute_bins(a, centroids):
    grid = _N // _RBLK
    cnorm = jnp.sum(centroids * centroids, axis=1)[None, :]
    return pl.pallas_call(
        _bins_body,
        grid=(grid,),
        in_specs=[
            pl.BlockSpec((_RBLK, _D), lambda i: (i, 0)),
            pl.BlockSpec((_K, _D), lambda i: (0, 0)),
            pl.BlockSpec((1, _K), lambda i: (0, 0)),
        ],
        out_specs=pl.BlockSpec((_RBLK,), lambda i: (i,)),
        out_shape=jax.ShapeDtypeStruct((_N,), jnp.int32),
    )(a, centroids, cnorm)


def _sc_residual_body(bins_hbm, act_hbm, ctr_hbm, out_hbm, idx_v, act_v, ctr_v, sem):
    wid = lax.axis_index("s") * _NC + lax.axis_index("c")
    base = wid * _RPW
    pltpu.sync_copy(bins_hbm.at[pl.ds(base, _RPW)], idx_v)

    def chunk(t, carry):
        rb = base + t * _CH
        pltpu.sync_copy(act_hbm.at[pl.ds(rb, _CH)], act_v)
        pltpu.async_copy(ctr_hbm.at[idx_v.at[pl.ds(t * _CH, _CH)]], ctr_v, sem).wait()

        def row(r, c2):
            for j in range(_D // _L):
                sl = pl.ds(j * _L, _L)
                act_v[r, sl] = act_v[r, sl] - ctr_v[r, sl]
            return c2

        lax.fori_loop(0, _CH, row, 0)
        pltpu.sync_copy(act_v, out_hbm.at[pl.ds(rb, _CH)])
        return carry

    lax.fori_loop(0, _RPW // _CH, chunk, 0)


@functools.lru_cache(maxsize=1)
def _sc_residual():
    return pl.kernel(
        _sc_residual_body,
        mesh=plsc.VectorSubcoreMesh(core_axis_name="c", subcore_axis_name="s"),
        out_type=jax.ShapeDtypeStruct((_N, _D), jnp.float32),
        scratch_types=[
            pltpu.VMEM((_RPW,), jnp.int32),
            pltpu.VMEM((_CH, _D), jnp.float32),
            pltpu.VMEM((_CH, 128), jnp.float32),
            pltpu.SemaphoreType.DMA,
        ],
    )


def kernel(action, centroids):
    B, T, D = action.shape
    a = action.reshape(B * T, D)
    bins = _compute_bins(a, centroids)
    cpad = jnp.pad(centroids, ((0, 0), (0, 128 - _D)))
    resid = _sc_residual()(bins, a, cpad)
    return bins.reshape(B, T, 1).astype(jnp.int64), resid.reshape(B, T, D)


# speedup vs baseline: 1.6329x; 1.1060x over previous
import functools

import jax
import jax.numpy as jnp
from jax import lax
from jax.experimental import pallas as pl
from jax.experimental.pallas import tpu as pltpu
from jax.experimental.pallas import tpu_sc as plsc

_K = 1024
_D = 64
_N = 32768
_RBLK = 1024

_NC = 2
_NS = 16
_NW = _NC * _NS
_RPW = _N // _NW
_CH = 128
_L = 16


def _bins_body(a_ref, c_ref, cn_ref, bins_ref):
    a = a_ref[...]
    c = c_ref[...]
    prod = lax.dot_general(
        a, c,
        dimension_numbers=(((1,), (1,)), ((), ())),
        preferred_element_type=jnp.float32,
    )
    sq = jnp.sum(a * a, axis=1, keepdims=True) - 2.0 * prod + cn_ref[...]
    dists = jnp.maximum(sq, 0.0)
    m = jnp.min(dists, axis=1, keepdims=True)
    idx = lax.broadcasted_iota(jnp.int32, dists.shape, 1).astype(jnp.float32)
    bins_ref[...] = jnp.min(jnp.where(dists <= m, idx, float(_K)), axis=1).astype(jnp.int32)


def _compute_bins(a, centroids):
    grid = _N // _RBLK
    cnorm = jnp.sum(centroids * centroids, axis=1)[None, :]
    return pl.pallas_call(
        _bins_body,
        grid=(grid,),
        in_specs=[
            pl.BlockSpec((_RBLK, _D), lambda i: (i, 0)),
            pl.BlockSpec((_K, _D), lambda i: (0, 0)),
            pl.BlockSpec((1, _K), lambda i: (0, 0)),
        ],
        out_specs=pl.BlockSpec((_RBLK,), lambda i: (i,)),
        out_shape=jax.ShapeDtypeStruct((_N,), jnp.int32),
    )(a, centroids, cnorm)


def _sc_residual_body(bins_hbm, act_hbm, ctr_hbm, out_hbm,
                      idx_v, a0, a1, g0, g1,
                      sa0, sa1, sg0, sg1, so0, so1):
    wid = lax.axis_index("s") * _NC + lax.axis_index("c")
    base = wid * _RPW
    pltpu.sync_copy(bins_hbm.at[pl.ds(base, _RPW)], idx_v)

    abufs, gbufs = (a0, a1), (g0, g1)
    sas, sgs, sos = (sa0, sa1), (sg0, sg1), (so0, so1)
    nch = _RPW // _CH
    in_h, out_h = {}, {}

    def start(t):
        b = t % 2
        rb = base + t * _CH
        ha = pltpu.async_copy(act_hbm.at[pl.ds(rb, _CH)], abufs[b], sas[b])
        hg = pltpu.async_copy(
            ctr_hbm.at[idx_v.at[pl.ds(t * _CH, _CH)]], gbufs[b], sgs[b])
        in_h[t] = (ha, hg)

    start(0)
    for t in range(nch):
        b = t % 2
        if t + 1 < nch:
            if t >= 1:
                out_h[t - 1].wait()
            start(t + 1)
        ha, hg = in_h.pop(t)
        ha.wait()
        hg.wait()
        ab, gb = abufs[b], gbufs[b]

        def rows(i, cc):
            r0 = i * 4
            for k in range(4):
                for j in range(_D // _L):
                    sl = pl.ds(j * _L, _L)
                    ab[r0 + k, sl] = ab[r0 + k, sl] - gb[r0 + k, sl]
            return cc

        lax.fori_loop(0, _CH // 4, rows, 0)
        out_h[t] = pltpu.async_copy(ab, out_hbm.at[pl.ds(base + t * _CH, _CH)], sos[b])

    out_h[nch - 2].wait()
    out_h[nch - 1].wait()


@functools.lru_cache(maxsize=1)
def _sc_residual():
    return pl.kernel(
        _sc_residual_body,
        mesh=plsc.VectorSubcoreMesh(core_axis_name="c", subcore_axis_name="s"),
        out_type=jax.ShapeDtypeStruct((_N, _D), jnp.float32),
        scratch_types=[
            pltpu.VMEM((_RPW,), jnp.int32),
            pltpu.VMEM((_CH, _D), jnp.float32),
            pltpu.VMEM((_CH, _D), jnp.float32),
            pltpu.VMEM((_CH, 128), jnp.float32),
            pltpu.VMEM((_CH, 128), jnp.float32),
            pltpu.SemaphoreType.DMA,
            pltpu.SemaphoreType.DMA,
            pltpu.SemaphoreType.DMA,
            pltpu.SemaphoreType.DMA,
            pltpu.SemaphoreType.DMA,
            pltpu.SemaphoreType.DMA,
        ],
    )


def kernel(action, centroids):
    B, T, D = action.shape
    a = action.reshape(B * T, D)
    bins = _compute_bins(a, centroids)
    cpad = jnp.pad(centroids, ((0, 0), (0, 128 - _D)))
    resid = _sc_residual()(bins, a, cpad)
    return bins.reshape(B, T, 1).astype(jnp.int64), resid.reshape(B, T, D)


# speedup vs baseline: 1.6587x; 1.0158x over previous
import functools

import jax
import jax.numpy as jnp
from jax import lax
from jax.experimental import pallas as pl
from jax.experimental.pallas import tpu as pltpu
from jax.experimental.pallas import tpu_sc as plsc

_K = 1024
_D = 64
_N = 32768
_RBLK = 2048

_NC = 2
_NS = 16
_NW = _NC * _NS
_RPW = _N // _NW
_CH = 128
_L = 16


def _bins_body(a_ref, c_ref, cn_ref, if_ref, bins_ref):
    a = a_ref[...]
    c = c_ref[...]
    prod = lax.dot_general(
        a, c,
        dimension_numbers=(((1,), (1,)), ((), ())),
        preferred_element_type=jnp.float32,
    )
    sq = jnp.sum(a * a, axis=1, keepdims=True) - 2.0 * prod + cn_ref[...]
    dists = jnp.maximum(sq, 0.0)
    m = jnp.min(dists, axis=1, keepdims=True)
    bins_ref[...] = jnp.min(jnp.where(dists <= m, if_ref[...], float(_K)), axis=1).astype(jnp.int32)


def _compute_bins(a, centroids):
    grid = _N // _RBLK
    cnorm = jnp.sum(centroids * centroids, axis=1)[None, :]
    iota_f = jnp.arange(_K, dtype=jnp.float32)[None, :]
    return pl.pallas_call(
        _bins_body,
        grid=(grid,),
        in_specs=[
            pl.BlockSpec((_RBLK, _D), lambda i: (i, 0)),
            pl.BlockSpec((_K, _D), lambda i: (0, 0)),
            pl.BlockSpec((1, _K), lambda i: (0, 0)),
            pl.BlockSpec((1, _K), lambda i: (0, 0)),
        ],
        out_specs=pl.BlockSpec((_RBLK,), lambda i: (i,)),
        out_shape=jax.ShapeDtypeStruct((_N,), jnp.int32),
    )(a, centroids, cnorm, iota_f)


def _sc_residual_body(bins_hbm, act_hbm, ctr_hbm, out_hbm,
                      idx_v, a0, a1, g0, g1,
                      sa0, sa1, sg0, sg1, so0, so1):
    wid = lax.axis_index("s") * _NC + lax.axis_index("c")
    base = wid * _RPW
    pltpu.sync_copy(bins_hbm.at[pl.ds(base, _RPW)], idx_v)

    abufs, gbufs = (a0, a1), (g0, g1)
    sas, sgs, sos = (sa0, sa1), (sg0, sg1), (so0, so1)
    nch = _RPW // _CH
    in_h, out_h = {}, {}

    def start(t):
        b = t % 2
        rb = base + t * _CH
        ha = pltpu.async_copy(act_hbm.at[pl.ds(rb, _CH)], abufs[b], sas[b])
        hg = pltpu.async_copy(
            ctr_hbm.at[idx_v.at[pl.ds(t * _CH, _CH)]], gbufs[b], sgs[b])
        in_h[t] = (ha, hg)

    start(0)
    for t in range(nch):
        b = t % 2
        if t + 1 < nch:
            if t >= 1:
                out_h[t - 1].wait()
            start(t + 1)
        ha, hg = in_h.pop(t)
        ha.wait()
        hg.wait()
        ab, gb = abufs[b], gbufs[b]

        def rows(i, cc):
            r0 = i * 4
            for k in range(4):
                for j in range(_D // _L):
                    sl = pl.ds(j * _L, _L)
                    ab[r0 + k, sl] = ab[r0 + k, sl] - gb[r0 + k, sl]
            return cc

        lax.fori_loop(0, _CH // 4, rows, 0)
        out_h[t] = pltpu.async_copy(ab, out_hbm.at[pl.ds(base + t * _CH, _CH)], sos[b])

    out_h[nch - 2].wait()
    out_h[nch - 1].wait()


@functools.lru_cache(maxsize=1)
def _sc_residual():
    return pl.kernel(
        _sc_residual_body,
        mesh=plsc.VectorSubcoreMesh(core_axis_name="c", subcore_axis_name="s"),
        out_type=jax.ShapeDtypeStruct((_N, _D), jnp.float32),
        scratch_types=[
            pltpu.VMEM((_RPW,), jnp.int32),
            pltpu.VMEM((_CH, _D), jnp.float32),
            pltpu.VMEM((_CH, _D), jnp.float32),
            pltpu.VMEM((_CH, 128), jnp.float32),
            pltpu.VMEM((_CH, 128), jnp.float32),
            pltpu.SemaphoreType.DMA,
            pltpu.SemaphoreType.DMA,
            pltpu.SemaphoreType.DMA,
            pltpu.SemaphoreType.DMA,
            pltpu.SemaphoreType.DMA,
            pltpu.SemaphoreType.DMA,
        ],
    )


def kernel(action, centroids):
    B, T, D = action.shape
    a = action.reshape(B * T, D)
    bins = _compute_bins(a, centroids)
    cpad = jnp.pad(centroids, ((0, 0), (0, 128 - _D)))
    resid = _sc_residual()(bins, a, cpad)
    return bins.reshape(B, T, 1).astype(jnp.int64), resid.reshape(B, T, D)


# speedup vs baseline: 1.8299x; 1.1032x over previous
import functools

import jax
import jax.numpy as jnp
from jax import lax
from jax.experimental import pallas as pl
from jax.experimental.pallas import tpu as pltpu
from jax.experimental.pallas import tpu_sc as plsc

_K = 1024
_D = 64
_N = 32768
_RBLK = 2048

_NC = 2
_NS = 16
_NW = _NC * _NS
_RPW = _N // _NW
_CH = 128
_L = 16


_KCH = 256


def _bins_body(a_ref, c2_ref, cn_ref, if_ref, bins_ref):
    a = a_ref[...]
    anorm = jnp.sum(a * a, axis=1, keepdims=True)
    m_run = jnp.full((_RBLK, 1), jnp.inf, jnp.float32)
    i_run = jnp.full((_RBLK, 1), float(_K), jnp.float32)
    for k0 in range(0, _K, _KCH):
        c2 = c2_ref[pl.ds(k0, _KCH), :]
        cn = cn_ref[:, pl.ds(k0, _KCH)]
        io = if_ref[:, pl.ds(k0, _KCH)]
        prod2 = lax.dot_general(
            a, c2, dimension_numbers=(((1,), (1,)), ((), ())),
            preferred_element_type=jnp.float32,
        )
        sq = anorm - prod2 + cn
        m_k = jnp.min(sq, axis=1, keepdims=True)
        i_k = jnp.min(jnp.where(sq <= m_k, io, float(_K)), axis=1, keepdims=True)
        better = m_k < m_run
        m_run = jnp.where(better, m_k, m_run)
        i_run = jnp.where(better, i_k, i_run)
    bins_ref[...] = i_run[:, 0].astype(jnp.int32)


def _compute_bins(a, centroids):
    grid = _N // _RBLK
    cnorm = jnp.sum(centroids * centroids, axis=1)[None, :]
    iota_f = jnp.arange(_K, dtype=jnp.float32)[None, :]
    return pl.pallas_call(
        _bins_body,
        grid=(grid,),
        in_specs=[
            pl.BlockSpec((_RBLK, _D), lambda i: (i, 0)),
            pl.BlockSpec((_K, _D), lambda i: (0, 0)),
            pl.BlockSpec((1, _K), lambda i: (0, 0)),
            pl.BlockSpec((1, _K), lambda i: (0, 0)),
        ],
        out_specs=pl.BlockSpec((_RBLK,), lambda i: (i,)),
        out_shape=jax.ShapeDtypeStruct((_N,), jnp.int32),
    )(a, centroids * 2.0, cnorm, iota_f)


def _sc_residual_body(bins_hbm, act_hbm, ctr_hbm, out_hbm,
                      idx_v, a0, a1, g0, g1,
                      sa0, sa1, sg0, sg1, so0, so1):
    wid = lax.axis_index("s") * _NC + lax.axis_index("c")
    base = wid * _RPW
    pltpu.sync_copy(bins_hbm.at[pl.ds(base, _RPW)], idx_v)

    abufs, gbufs = (a0, a1), (g0, g1)
    sas, sgs, sos = (sa0, sa1), (sg0, sg1), (so0, so1)
    nch = _RPW // _CH
    in_h, out_h = {}, {}

    def start(t):
        b = t % 2
        rb = base + t * _CH
        ha = pltpu.async_copy(act_hbm.at[pl.ds(rb, _CH)], abufs[b], sas[b])
        hg = pltpu.async_copy(
            ctr_hbm.at[idx_v.at[pl.ds(t * _CH, _CH)]], gbufs[b], sgs[b])
        in_h[t] = (ha, hg)

    start(0)
    for t in range(nch):
        b = t % 2
        if t + 1 < nch:
            if t >= 1:
                out_h[t - 1].wait()
            start(t + 1)
        ha, hg = in_h.pop(t)
        ha.wait()
        hg.wait()
        ab, gb = abufs[b], gbufs[b]

        def rows(i, cc):
            r0 = i * 4
            for k in range(4):
                for j in range(_D // _L):
                    sl = pl.ds(j * _L, _L)
                    ab[r0 + k, sl] = ab[r0 + k, sl] - gb[r0 + k, sl]
            return cc

        lax.fori_loop(0, _CH // 4, rows, 0)
        out_h[t] = pltpu.async_copy(ab, out_hbm.at[pl.ds(base + t * _CH, _CH)], sos[b])

    out_h[nch - 2].wait()
    out_h[nch - 1].wait()


@functools.lru_cache(maxsize=1)
def _sc_residual():
    return pl.kernel(
        _sc_residual_body,
        mesh=plsc.VectorSubcoreMesh(core_axis_name="c", subcore_axis_name="s"),
        out_type=jax.ShapeDtypeStruct((_N, _D), jnp.float32),
        scratch_types=[
            pltpu.VMEM((_RPW,), jnp.int32),
            pltpu.VMEM((_CH, _D), jnp.float32),
            pltpu.VMEM((_CH, _D), jnp.float32),
            pltpu.VMEM((_CH, 128), jnp.float32),
            pltpu.VMEM((_CH, 128), jnp.float32),
            pltpu.SemaphoreType.DMA,
            pltpu.SemaphoreType.DMA,
            pltpu.SemaphoreType.DMA,
            pltpu.SemaphoreType.DMA,
            pltpu.SemaphoreType.DMA,
            pltpu.SemaphoreType.DMA,
        ],
    )


def kernel(action, centroids):
    B, T, D = action.shape
    a = action.reshape(B * T, D)
    bins = _compute_bins(a, centroids)
    cpad = jnp.pad(centroids, ((0, 0), (0, 128 - _D)))
    resid = _sc_residual()(bins, a, cpad)
    return bins.reshape(B, T, 1).astype(jnp.int64), resid.reshape(B, T, D)
